# baseline (speedup 1.0000x reference)
import jax
import jax.numpy as jnp
from jax import lax
from jax.experimental import pallas as pl
from jax.experimental.pallas import tpu as pltpu
from jax.experimental.pallas import tpu_sc as plsc

N_NODES = 10000
D_FEAT = 128
LANES = 16
NUM_CORES = 2
NUM_SUBCORES = 16
NUM_WORKERS = NUM_CORES * NUM_SUBCORES
CHUNK = 128
RB = 10
N_ACC = 10240
ROWS_PER_SUBCORE = N_ACC // NUM_SUBCORES
_CORE_SPLIT = (0, 16)


def _sc_aggregate(feat, idx5, g0, g1, with_deg):
    out_type = [jax.ShapeDtypeStruct((NUM_CORES, N_ACC, D_FEAT),
                                     jnp.float32)]
    if with_deg:
        out_type.append(
            jax.ShapeDtypeStruct((NUM_WORKERS, N_ACC), jnp.float32))

    scratch = [
        pltpu.VMEM_SHARED((N_ACC, D_FEAT), jnp.float32),
        pltpu.VMEM((2 * RB * 2, CHUNK), jnp.int32),
        pltpu.VMEM((CHUNK, D_FEAT), jnp.float32),
        pltpu.VMEM((CHUNK, D_FEAT), jnp.float32),
        pltpu.SemaphoreType.DMA,
        pltpu.SemaphoreType.DMA,
        pltpu.SemaphoreType.DMA,
        pltpu.SemaphoreType.DMA,
    ]
    if with_deg:
        scratch.append(pltpu.VMEM((N_ACC,), jnp.float32))

    def body(feat_hbm, idx_hbm, *rest):
        if with_deg:
            (acc_out, deg_out, acc_sh, ring, rows0, rows1,
             rsem0, rsem1, gsem0, gsem1, deg_v) = rest
        else:
            (acc_out, acc_sh, ring, rows0, rows1,
             rsem0, rsem1, gsem0, gsem1) = rest
        rsems = (rsem0, rsem1)

        cid = lax.axis_index("c")
        sid = lax.axis_index("s")
        wid = sid * NUM_CORES + cid

        gt = lax.select(cid == 0, jnp.int32(g0), jnp.int32(g1))

        @pl.when(gt > 0)
        def _():
            pltpu.async_copy(idx_hbm.at[wid, 0],
                             ring.at[pl.ds(0, RB * 2)], rsem0)

        @pl.when(gt > 1)
        def _():
            pltpu.async_copy(idx_hbm.at[wid, 1],
                             ring.at[pl.ds(RB * 2, RB * 2)], rsem1)

        zeros16 = jnp.zeros((LANES,), jnp.float32)

        def zbody(i, carry):
            for j in range(D_FEAT // LANES):
                rows0[i, pl.ds(j * LANES, LANES)] = zeros16
            return carry

        lax.fori_loop(0, CHUNK, zbody, 0)

        if with_deg:
            def zdeg(i, carry):
                deg_v[pl.ds(i * LANES, LANES)] = zeros16
                return carry

            lax.fori_loop(0, N_ACC // LANES, zdeg, 0)

        base = sid * ROWS_PER_SUBCORE
        for t in range(ROWS_PER_SUBCORE // CHUNK):
            pltpu.sync_copy(rows0, acc_sh.at[pl.ds(base + t * CHUNK, CHUNK)])

        plsc.subcore_barrier()

        def histogram(r):
            for k in range(CHUNK // LANES):
                d16 = ring[r, pl.ds(k * LANES, LANES)]
                cnt, last = plsc.scan_count(d16)
                plsc.addupdate_scatter(
                    deg_v, [d16], cnt.astype(jnp.float32), mask=last)

        def run_group(g, hb, rsem):
            pltpu.make_async_copy(idx_hbm.at[wid, g],
                                  ring.at[pl.ds(hb, RB * 2)], rsem).wait()
            pltpu.async_copy(feat_hbm.at[ring.at[hb]], rows0, gsem0)

            def pair_body(p, carry):
                j = 2 * p
                pltpu.async_copy(feat_hbm.at[ring.at[hb + (j + 1) * 2]],
                                 rows1, gsem1)
                if with_deg:
                    histogram(hb + j * 2 + 1)
                pltpu.make_async_copy(feat_hbm.at[ring.at[hb + j * 2]],
                                      rows0, gsem0).wait()
                pltpu.sync_copy(rows0, acc_sh.at[ring.at[hb + j * 2 + 1]],
                                add=True)

                @pl.when(p < RB // 2 - 1)
                def _():
                    pltpu.async_copy(
                        feat_hbm.at[ring.at[hb + (j + 2) * 2]],
                        rows0, gsem0)

                if with_deg:
                    histogram(hb + (j + 1) * 2 + 1)
                pltpu.make_async_copy(feat_hbm.at[ring.at[hb + (j + 1) * 2]],
                                      rows1, gsem1).wait()
                pltpu.sync_copy(rows1,
                                acc_sh.at[ring.at[hb + (j + 1) * 2 + 1]],
                                add=True)
                return carry

            lax.fori_loop(0, RB // 2, pair_body, 0)
            @pl.when(g + 2 < gt)
            def _():
                pltpu.async_copy(idx_hbm.at[wid, g + 2],
                                 ring.at[pl.ds(hb, RB * 2)], rsem)

        def group_pair(go, carry):
            run_group(2 * go, 0, rsem0)
            run_group(2 * go + 1, RB * 2, rsem1)
            return carry

        lax.fori_loop(0, gt // 2, group_pair, 0)

        plsc.subcore_barrier()

        pltpu.sync_copy(acc_sh.at[pl.ds(base, ROWS_PER_SUBCORE)],
                        acc_out.at[cid, pl.ds(base, ROWS_PER_SUBCORE)])
        if with_deg:
            pltpu.sync_copy(deg_v, deg_out.at[wid])

    mesh = plsc.VectorSubcoreMesh(core_axis_name="c", subcore_axis_name="s")
    fn = pl.kernel(body, out_type=out_type, mesh=mesh, scratch_types=scratch,
                   compiler_params=pltpu.CompilerParams(
                       needs_layout_passes=False))
    return fn(feat, idx5)


def _tc_layer(xx, accp, degp, W_self, W_neigh, b, relu):
    R = 256
    grid = (pl.cdiv(N_NODES, R),)

    def body(x_ref, a_ref, d_ref, ws_ref, wn_ref, b_ref, o_ref):
        acc = a_ref[0] + a_ref[1]
        deg = jnp.sum(d_ref[...], axis=0)
        hn = acc / jnp.maximum(deg, 1.0)[:, None]
        out = (jnp.dot(x_ref[...], ws_ref[...],
                       preferred_element_type=jnp.float32)
               + jnp.dot(hn, wn_ref[...], preferred_element_type=jnp.float32)
               + b_ref[...])
        if relu:
            out = jnp.maximum(out, 0.0)
        o_ref[...] = out

    return pl.pallas_call(
        body,
        grid=grid,
        in_specs=[
            pl.BlockSpec((R, D_FEAT), lambda i: (i, 0)),
            pl.BlockSpec((NUM_CORES, R, D_FEAT), lambda i: (0, i, 0)),
            pl.BlockSpec((NUM_WORKERS, R), lambda i: (0, i)),
            pl.BlockSpec((D_FEAT, D_FEAT), lambda i: (0, 0)),
            pl.BlockSpec((D_FEAT, D_FEAT), lambda i: (0, 0)),
            pl.BlockSpec((1, D_FEAT), lambda i: (0, 0)),
        ],
        out_specs=pl.BlockSpec((R, D_FEAT), lambda i: (i, 0)),
        out_shape=jax.ShapeDtypeStruct((N_NODES, D_FEAT), jnp.float32),
    )(xx, accp, degp, W_self, W_neigh, b.reshape(1, D_FEAT))


def kernel(x, edge_index, W_self1, W_neigh1, b1, W1, bW1, W2, bW2,
           W_self2, W_neigh2, b2):
    e = edge_index.shape[1]
    src = edge_index[0].astype(jnp.int32)
    dst = edge_index[1].astype(jnp.int32)

    gpe = RB * CHUNK
    total_groups = (e + gpe - 1) // gpe
    per_worker = (total_groups + NUM_SUBCORES - 1) // NUM_SUBCORES
    g0, g1 = _CORE_SPLIT
    assert g0 + g1 == per_worker and g0 % 2 == 0 and g1 % 2 == 0
    c0_edges = NUM_SUBCORES * g0 * gpe
    e_pad = NUM_SUBCORES * (g0 + g1) * gpe
    pad = e_pad - e
    src_p = jnp.concatenate([src, jnp.zeros((pad,), jnp.int32)])
    pad_dst = N_NODES + (jnp.arange(pad, dtype=jnp.int32)
                         % (N_ACC - N_NODES))
    dst_p = jnp.concatenate([dst, pad_dst])

    gmax = max(g0, g1, 2)

    def to_worker_layout(a):
        a0 = a[:c0_edges].reshape(NUM_SUBCORES, g0, RB, CHUNK)
        a1 = a[c0_edges:].reshape(NUM_SUBCORES, g1, RB, CHUNK)
        z = jnp.zeros((NUM_SUBCORES, gmax, RB, CHUNK), jnp.int32)
        a0 = jnp.concatenate([a0, z[:, g0:]], axis=1)
        a1 = jnp.concatenate([a1, z[:, g1:]], axis=1)
        return jnp.stack([a0, a1], axis=1)

    idx5 = jnp.stack([to_worker_layout(src_p), to_worker_layout(dst_p)],
                     axis=4).reshape(NUM_WORKERS, gmax, RB * 2, CHUNK)

    acc1, deg = _sc_aggregate(x, idx5, g0, g1, with_deg=True)
    h = _tc_layer(x, acc1, deg, W_self1, W_neigh1, b1, relu=True)
    (acc2,) = _sc_aggregate(h, idx5, g0, g1, with_deg=False)
    h2 = _tc_layer(h, acc2, deg, W_self2, W_neigh2, b2, relu=False)
    return h2


# speedup vs baseline: 1.5115x; 1.5115x over previous
import jax
import jax.numpy as jnp
from jax import lax
from jax.experimental import pallas as pl
from jax.experimental.pallas import tpu as pltpu
from jax.experimental.pallas import tpu_sc as plsc

N_NODES = 10000
D_FEAT = 128
LANES = 16
NUM_CORES = 2
NUM_SUBCORES = 16
NUM_WORKERS = NUM_CORES * NUM_SUBCORES
CHUNK = 128
N_ACC = 10240
ROWS_PER_SUBCORE = N_ACC // NUM_SUBCORES


def _sc_aggregate(feat, src3, dst3, with_deg):
    ch = src3.shape[1]

    out_type = [jax.ShapeDtypeStruct((NUM_CORES, N_ACC, D_FEAT),
                                     jnp.float32)]
    if with_deg:
        out_type.append(
            jax.ShapeDtypeStruct((NUM_WORKERS, N_ACC), jnp.float32))

    scratch = [
        pltpu.VMEM_SHARED((N_ACC, D_FEAT), jnp.float32),
        pltpu.VMEM((ch, CHUNK), jnp.int32),
        pltpu.VMEM((ch, CHUNK), jnp.int32),
        pltpu.VMEM((CHUNK, D_FEAT), jnp.float32),
        pltpu.SemaphoreType.DMA,
    ]
    if with_deg:
        scratch.append(pltpu.VMEM((N_ACC,), jnp.float32))

    def body(feat_hbm, src_hbm, dst_hbm, *rest):
        if with_deg:
            acc_out, deg_out, acc_sh, src_v, dst_v, rows_v, sem, deg_v = rest
        else:
            acc_out, acc_sh, src_v, dst_v, rows_v, sem = rest

        cid = lax.axis_index("c")
        sid = lax.axis_index("s")
        wid = sid * NUM_CORES + cid

        zeros16 = jnp.zeros((LANES,), jnp.float32)

        def zbody(i, carry):
            for j in range(D_FEAT // LANES):
                rows_v[i, pl.ds(j * LANES, LANES)] = zeros16
            return carry

        lax.fori_loop(0, CHUNK, zbody, 0)

        if with_deg:
            def zdeg(i, carry):
                deg_v[pl.ds(i * LANES, LANES)] = zeros16
                return carry

            lax.fori_loop(0, N_ACC // LANES, zdeg, 0)

        base = sid * ROWS_PER_SUBCORE
        for t in range(ROWS_PER_SUBCORE // CHUNK):
            pltpu.sync_copy(rows_v, acc_sh.at[pl.ds(base + t * CHUNK, CHUNK)])

        pltpu.sync_copy(src_hbm.at[wid], src_v)
        pltpu.sync_copy(dst_hbm.at[wid], dst_v)

        plsc.subcore_barrier()

        def chunk_body(j, carry):
            gather = pltpu.async_copy(feat_hbm.at[src_v.at[j]], rows_v, sem)
            if with_deg:
                for k in range(CHUNK // LANES):
                    d16 = dst_v[j, pl.ds(k * LANES, LANES)]
                    cnt, last = plsc.scan_count(d16)
                    plsc.addupdate_scatter(
                        deg_v, [d16], cnt.astype(jnp.float32), mask=last)
            gather.wait()
            pltpu.sync_copy(rows_v, acc_sh.at[dst_v.at[j]], add=True)
            return carry

        lax.fori_loop(0, ch, chunk_body, 0)

        plsc.subcore_barrier()

        pltpu.sync_copy(acc_sh.at[pl.ds(base, ROWS_PER_SUBCORE)],
                        acc_out.at[cid, pl.ds(base, ROWS_PER_SUBCORE)])
        if with_deg:
            pltpu.sync_copy(deg_v, deg_out.at[wid])

    mesh = plsc.VectorSubcoreMesh(core_axis_name="c", subcore_axis_name="s")
    fn = pl.kernel(body, out_type=out_type, mesh=mesh, scratch_types=scratch,
                   compiler_params=pltpu.CompilerParams(
                       needs_layout_passes=False))
    return fn(feat, src3, dst3)


def _tc_layer(xx, accp, degp, W_self, W_neigh, b, relu):
    R = 256
    grid = (pl.cdiv(N_NODES, R),)

    def body(x_ref, a_ref, d_ref, ws_ref, wn_ref, b_ref, o_ref):
        acc = a_ref[0] + a_ref[1]
        deg = jnp.sum(d_ref[...], axis=0)
        hn = acc / jnp.maximum(deg, 1.0)[:, None]
        out = (jnp.dot(x_ref[...], ws_ref[...],
                       preferred_element_type=jnp.float32)
               + jnp.dot(hn, wn_ref[...], preferred_element_type=jnp.float32)
               + b_ref[...])
        if relu:
            out = jnp.maximum(out, 0.0)
        o_ref[...] = out

    return pl.pallas_call(
        body,
        grid=grid,
        in_specs=[
            pl.BlockSpec((R, D_FEAT), lambda i: (i, 0)),
            pl.BlockSpec((NUM_CORES, R, D_FEAT), lambda i: (0, i, 0)),
            pl.BlockSpec((NUM_WORKERS, R), lambda i: (0, i)),
            pl.BlockSpec((D_FEAT, D_FEAT), lambda i: (0, 0)),
            pl.BlockSpec((D_FEAT, D_FEAT), lambda i: (0, 0)),
            pl.BlockSpec((1, D_FEAT), lambda i: (0, 0)),
        ],
        out_specs=pl.BlockSpec((R, D_FEAT), lambda i: (i, 0)),
        out_shape=jax.ShapeDtypeStruct((N_NODES, D_FEAT), jnp.float32),
    )(xx, accp, degp, W_self, W_neigh, b.reshape(1, D_FEAT))


def kernel(x, edge_index, W_self1, W_neigh1, b1, W1, bW1, W2, bW2,
           W_self2, W_neigh2, b2):
    e = edge_index.shape[1]
    src = edge_index[0].astype(jnp.int32)
    dst = edge_index[1].astype(jnp.int32)

    epw = NUM_WORKERS * CHUNK
    e_pad = ((e + epw - 1) // epw) * epw
    pad = e_pad - e
    src_p = jnp.concatenate([src, jnp.zeros((pad,), jnp.int32)])
    pad_dst = N_NODES + (jnp.arange(pad, dtype=jnp.int32)
                         % (N_ACC - N_NODES))
    dst_p = jnp.concatenate([dst, pad_dst])
    ch = e_pad // epw
    src3 = src_p.reshape(NUM_WORKERS, ch, CHUNK)
    dst3 = dst_p.reshape(NUM_WORKERS, ch, CHUNK)

    acc1, deg = _sc_aggregate(x, src3, dst3, with_deg=True)
    h = _tc_layer(x, acc1, deg, W_self1, W_neigh1, b1, relu=True)
    (acc2,) = _sc_aggregate(h, src3, dst3, with_deg=False)
    h2 = _tc_layer(h, acc2, deg, W_self2, W_neigh2, b2, relu=False)
    return h2


# speedup vs baseline: 1.5166x; 1.0034x over previous
import jax
import jax.numpy as jnp
from jax import lax
from jax.experimental import pallas as pl
from jax.experimental.pallas import tpu as pltpu
from jax.experimental.pallas import tpu_sc as plsc

N_NODES = 10000
D_FEAT = 128
LANES = 16
NUM_CORES = 2
NUM_SUBCORES = 16
NUM_WORKERS = NUM_CORES * NUM_SUBCORES
CHUNK = 128
N_ACC = 10240
ROWS_PER_SUBCORE = N_ACC // NUM_SUBCORES


def _sc_aggregate(feat, src3, dst3, with_deg):
    ch = src3.shape[1]

    out_type = [jax.ShapeDtypeStruct((NUM_CORES, N_ACC, D_FEAT),
                                     jnp.float32)]
    if with_deg:
        out_type.append(
            jax.ShapeDtypeStruct((NUM_WORKERS, N_ACC), jnp.float32))

    scratch = [
        pltpu.VMEM_SHARED((N_ACC, D_FEAT), jnp.float32),
        pltpu.VMEM((ch, CHUNK), jnp.int32),
        pltpu.VMEM((ch, CHUNK), jnp.int32),
        pltpu.VMEM((CHUNK, D_FEAT), jnp.float32),
        pltpu.SemaphoreType.DMA,
    ]
    if with_deg:
        scratch.append(pltpu.VMEM((N_ACC,), jnp.float32))

    def body(feat_hbm, src_hbm, dst_hbm, *rest):
        if with_deg:
            acc_out, deg_out, acc_sh, src_v, dst_v, rows_v, sem, deg_v = rest
        else:
            acc_out, acc_sh, src_v, dst_v, rows_v, sem = rest

        cid = lax.axis_index("c")
        sid = lax.axis_index("s")
        wid = sid * NUM_CORES + cid

        zeros16 = jnp.zeros((LANES,), jnp.float32)

        def zbody(i, carry):
            for j in range(D_FEAT // LANES):
                rows_v[i, pl.ds(j * LANES, LANES)] = zeros16
            return carry

        lax.fori_loop(0, CHUNK, zbody, 0)

        if with_deg:
            def zdeg(i, carry):
                deg_v[pl.ds(i * LANES, LANES)] = zeros16
                return carry

            lax.fori_loop(0, N_ACC // LANES, zdeg, 0)

        base = sid * ROWS_PER_SUBCORE
        for t in range(ROWS_PER_SUBCORE // CHUNK):
            pltpu.sync_copy(rows_v, acc_sh.at[pl.ds(base + t * CHUNK, CHUNK)])

        pltpu.sync_copy(src_hbm.at[wid], src_v)
        pltpu.sync_copy(dst_hbm.at[wid], dst_v)

        plsc.subcore_barrier()

        def chunk_body(j, carry):
            half = CHUNK // 2
            g0 = pltpu.async_copy(
                feat_hbm.at[src_v.at[j, pl.ds(0, half)]],
                rows_v.at[pl.ds(0, half)], sem)
            g1 = pltpu.async_copy(
                feat_hbm.at[src_v.at[j, pl.ds(half, half)]],
                rows_v.at[pl.ds(half, half)], sem)
            if with_deg:
                for k in range(CHUNK // LANES):
                    d16 = dst_v[j, pl.ds(k * LANES, LANES)]
                    cnt, last = plsc.scan_count(d16)
                    plsc.addupdate_scatter(
                        deg_v, [d16], cnt.astype(jnp.float32), mask=last)
            g0.wait()
            g1.wait()
            pltpu.sync_copy(rows_v, acc_sh.at[dst_v.at[j]], add=True)
            return carry

        lax.fori_loop(0, ch, chunk_body, 0)

        plsc.subcore_barrier()

        pltpu.sync_copy(acc_sh.at[pl.ds(base, ROWS_PER_SUBCORE)],
                        acc_out.at[cid, pl.ds(base, ROWS_PER_SUBCORE)])
        if with_deg:
            pltpu.sync_copy(deg_v, deg_out.at[wid])

    mesh = plsc.VectorSubcoreMesh(core_axis_name="c", subcore_axis_name="s")
    fn = pl.kernel(body, out_type=out_type, mesh=mesh, scratch_types=scratch,
                   compiler_params=pltpu.CompilerParams(
                       needs_layout_passes=False))
    return fn(feat, src3, dst3)


def _tc_layer(xx, accp, degp, W_self, W_neigh, b, relu):
    R = 256
    grid = (pl.cdiv(N_NODES, R),)

    def body(x_ref, a_ref, d_ref, ws_ref, wn_ref, b_ref, o_ref):
        acc = a_ref[0] + a_ref[1]
        deg = jnp.sum(d_ref[...], axis=0)
        hn = acc / jnp.maximum(deg, 1.0)[:, None]
        out = (jnp.dot(x_ref[...], ws_ref[...],
                       preferred_element_type=jnp.float32)
               + jnp.dot(hn, wn_ref[...], preferred_element_type=jnp.float32)
               + b_ref[...])
        if relu:
            out = jnp.maximum(out, 0.0)
        o_ref[...] = out

    return pl.pallas_call(
        body,
        grid=grid,
        in_specs=[
            pl.BlockSpec((R, D_FEAT), lambda i: (i, 0)),
            pl.BlockSpec((NUM_CORES, R, D_FEAT), lambda i: (0, i, 0)),
            pl.BlockSpec((NUM_WORKERS, R), lambda i: (0, i)),
            pl.BlockSpec((D_FEAT, D_FEAT), lambda i: (0, 0)),
            pl.BlockSpec((D_FEAT, D_FEAT), lambda i: (0, 0)),
            pl.BlockSpec((1, D_FEAT), lambda i: (0, 0)),
        ],
        out_specs=pl.BlockSpec((R, D_FEAT), lambda i: (i, 0)),
        out_shape=jax.ShapeDtypeStruct((N_NODES, D_FEAT), jnp.float32),
    )(xx, accp, degp, W_self, W_neigh, b.reshape(1, D_FEAT))


def kernel(x, edge_index, W_self1, W_neigh1, b1, W1, bW1, W2, bW2,
           W_self2, W_neigh2, b2):
    e = edge_index.shape[1]
    src = edge_index[0].astype(jnp.int32)
    dst = edge_index[1].astype(jnp.int32)

    epw = NUM_WORKERS * CHUNK
    e_pad = ((e + epw - 1) // epw) * epw
    pad = e_pad - e
    src_p = jnp.concatenate([src, jnp.zeros((pad,), jnp.int32)])
    pad_dst = N_NODES + (jnp.arange(pad, dtype=jnp.int32)
                         % (N_ACC - N_NODES))
    dst_p = jnp.concatenate([dst, pad_dst])
    ch = e_pad // epw
    src3 = src_p.reshape(NUM_WORKERS, ch, CHUNK)
    dst3 = dst_p.reshape(NUM_WORKERS, ch, CHUNK)

    acc1, deg = _sc_aggregate(x, src3, dst3, with_deg=True)
    h = _tc_layer(x, acc1, deg, W_self1, W_neigh1, b1, relu=True)
    (acc2,) = _sc_aggregate(h, src3, dst3, with_deg=False)
    h2 = _tc_layer(h, acc2, deg, W_self2, W_neigh2, b2, relu=False)
    return h2
